# Initial kernel scaffold; baseline (speedup 1.0000x reference)
import jax
import jax.numpy as jnp
from jax.experimental import pallas as pl


def kernel(x, edge_index, batch, W1, b1, W2, b2, fc1_w, fc1_b, fc2_w, fc2_b):
    raise NotImplementedError("write your pallas kernel here")


# speedup vs baseline: 19.9052x; 19.9052x over previous
import functools

import jax
import jax.numpy as jnp
from jax import lax
from jax.experimental import pallas as pl
from jax.experimental.pallas import tpu as pltpu
from jax.experimental.pallas import tpu_sc as plsc

N = 10000
E = 320000
NP = 10240
G = 64
NC = 2
NS = 16
NW = NC * NS
EPT = E // NW
CH = 128
NCHUNK = E // CH
CPW = NCHUNK // NW
CREM = NCHUNK - CPW * NW
RPT = NP // NS

_mesh = functools.partial(
    plsc.VectorSubcoreMesh, core_axis_name="c", subcore_axis_name="s"
)


def _deg_body(dst_hbm, out_hbm, idx_v, deg_v):
    c = lax.axis_index("c")
    s = lax.axis_index("s")
    wid = c * NS + s

    def zero(i, _):
        deg_v[pl.ds(i * 16, 16)] = jnp.zeros((16,), jnp.float32)
        return 0

    lax.fori_loop(0, NP // 16, zero, 0)

    pltpu.sync_copy(dst_hbm.at[pl.ds(wid * EPT, EPT)], idx_v)
    ones = jnp.ones((16,), jnp.float32)

    def body(j, _):
        idx = idx_v[pl.ds(j * 16, 16)]
        plsc.addupdate_scatter(deg_v, [idx], ones)
        return 0

    lax.fori_loop(0, EPT // 16, body, 0)
    pltpu.sync_copy(deg_v, out_hbm.at[wid])


def _deg_call(dst):
    return pl.kernel(
        _deg_body,
        out_type=jax.ShapeDtypeStruct((NW, NP), jnp.float32),
        mesh=_mesh(),
        scratch_types=[
            pltpu.VMEM((EPT,), jnp.int32),
            pltpu.VMEM((NP,), jnp.float32),
        ],
        compiler_params=pltpu.CompilerParams(needs_layout_passes=False),
    )(dst)


def _spmm_body(F, xs_hbm, src_hbm, dst_hbm, out_hbm, idx_s, idx_d, rows, acc, sem):
    c = lax.axis_index("c")
    s = lax.axis_index("s")
    wid = c * NS + s

    def zr(r, _):
        def zc(k, _):
            rows[r, pl.ds(k * 16, 16)] = jnp.zeros((16,), jnp.float32)
            return 0

        lax.fori_loop(0, F // 16, zc, 0)
        return 0

    lax.fori_loop(0, CH, zr, 0)
    for j in range(RPT // CH):
        pltpu.sync_copy(rows, acc.at[pl.ds(s * RPT + j * CH, CH)])
    plsc.subcore_barrier()

    base = wid * CPW + jnp.minimum(wid, CREM)
    n = jnp.where(wid < CREM, CPW + 1, CPW)

    def edge(i, _):
        off = (base + i) * CH
        pltpu.sync_copy(src_hbm.at[pl.ds(off, CH)], idx_s)
        pltpu.async_copy(xs_hbm.at[idx_s], rows, sem).wait()
        pltpu.sync_copy(dst_hbm.at[pl.ds(off, CH)], idx_d)
        pltpu.sync_copy(rows, acc.at[idx_d], add=True)
        return 0

    lax.fori_loop(0, n, edge, 0)
    plsc.subcore_barrier()
    pltpu.sync_copy(
        acc.at[pl.ds(s * RPT, RPT)], out_hbm.at[c, pl.ds(s * RPT, RPT)]
    )


def _spmm_call(F, xs, src, dst):
    return pl.kernel(
        functools.partial(_spmm_body, F),
        out_type=jax.ShapeDtypeStruct((NC, NP, F), jnp.float32),
        mesh=_mesh(),
        scratch_types=[
            pltpu.VMEM((CH,), jnp.int32),
            pltpu.VMEM((CH,), jnp.int32),
            pltpu.VMEM((CH, F), jnp.float32),
            pltpu.VMEM_SHARED((NP, F), jnp.float32),
            pltpu.SemaphoreType.DMA,
        ],
        compiler_params=pltpu.CompilerParams(use_tc_tiling_on_sc=False),
    )(xs, src, dst)


def _dinv_body(degp_ref, dinv_ref):
    deg = 1.0 + jnp.sum(degp_ref[...], axis=0, keepdims=True)
    dinv_ref[...] = lax.rsqrt(jnp.maximum(deg, 1e-12))


def _scale_body(x_ref, d_ref, o_ref):
    o_ref[...] = x_ref[...] * d_ref[...]


def _mid_body(p0, p1, xs, d, w1, b1, w2, o):
    agg = d[...] * (p0[...] + p1[...] + xs[...])
    h1 = jnp.maximum(
        jnp.dot(agg, w1[...], preferred_element_type=jnp.float32) + b1[...], 0.0
    )
    g = jnp.dot(h1, w2[...], preferred_element_type=jnp.float32)
    o[...] = g * d[...]


def _head_body(q0, q1, gs, d, b2, bt, fc1w, fc1b, fc2w, fc2b, o):
    h2 = jnp.maximum(d[...] * (q0[...] + q1[...] + gs[...]) + b2[...], 0.0)
    gid = lax.broadcasted_iota(jnp.int32, (G, NP), 0)
    oh = (gid == bt[...]).astype(jnp.float32)
    psum = jnp.dot(oh, h2, preferred_element_type=jnp.float32)
    cnt = jnp.sum(oh, axis=1, keepdims=True)
    pooled = psum / jnp.maximum(cnt, 1.0)
    z = jnp.tanh(jnp.dot(pooled, fc1w[...], preferred_element_type=jnp.float32) + fc1b[...])
    zz = jnp.dot(z, fc2w[...], preferred_element_type=jnp.float32) + fc2b[...]
    o[...] = jax.nn.sigmoid(zz)


def kernel(x, edge_index, batch, W1, b1, W2, b2, fc1_w, fc1_b, fc2_w, fc2_b):
    f32 = jnp.float32
    src = edge_index[0].astype(jnp.int32)
    dst = edge_index[1].astype(jnp.int32)
    x_pad = jnp.pad(x.astype(f32), ((0, NP - N), (0, 0)))
    batch_pad = jnp.pad(
        batch.astype(jnp.int32), (0, NP - N), constant_values=2**20
    ).reshape(1, NP)

    degp = _deg_call(dst)

    dinv_row = pl.pallas_call(
        _dinv_body,
        out_shape=jax.ShapeDtypeStruct((1, NP), f32),
    )(degp)
    dinv_col = dinv_row.reshape(NP, 1)

    RB = 1280
    xs = pl.pallas_call(
        _scale_body,
        grid=(NP // RB,),
        in_specs=[
            pl.BlockSpec((RB, 128), lambda i: (i, 0)),
            pl.BlockSpec((RB, 1), lambda i: (i, 0)),
        ],
        out_specs=pl.BlockSpec((RB, 128), lambda i: (i, 0)),
        out_shape=jax.ShapeDtypeStruct((NP, 128), f32),
    )(x_pad, dinv_col)

    P = _spmm_call(128, xs, src, dst)

    gs = pl.pallas_call(
        _mid_body,
        grid=(NP // RB,),
        in_specs=[
            pl.BlockSpec((RB, 128), lambda i: (i, 0)),
            pl.BlockSpec((RB, 128), lambda i: (i, 0)),
            pl.BlockSpec((RB, 128), lambda i: (i, 0)),
            pl.BlockSpec((RB, 1), lambda i: (i, 0)),
            pl.BlockSpec((128, 128), lambda i: (0, 0)),
            pl.BlockSpec((1, 128), lambda i: (0, 0)),
            pl.BlockSpec((128, 32), lambda i: (0, 0)),
        ],
        out_specs=pl.BlockSpec((RB, 32), lambda i: (i, 0)),
        out_shape=jax.ShapeDtypeStruct((NP, 32), f32),
    )(P[0], P[1], xs, dinv_col, W1, b1.reshape(1, 128), W2)

    Q = _spmm_call(32, gs, src, dst)

    out = pl.pallas_call(
        _head_body,
        out_shape=jax.ShapeDtypeStruct((G, 1), f32),
    )(
        Q[0],
        Q[1],
        gs,
        dinv_col,
        b2.reshape(1, 32),
        batch_pad,
        fc1_w,
        fc1_b.reshape(1, 16),
        fc2_w,
        fc2_b.reshape(1, 1),
    )
    return out
